# baseline (speedup 1.0000x reference)
import functools

import jax
import jax.numpy as jnp
from jax import lax
from jax.experimental import pallas as pl
from jax.experimental.pallas import tpu as pltpu
from jax.experimental.pallas import tpu_sc as plsc

STATE = 10
KM_ITERS = 10
TOPK = 3
PAD = 16
EPS = 1e-12
_HI = jax.lax.Precision.HIGHEST


GROUP = 2


def _tc_body(len_ref, feat_ref, lab_ref, out_ref):
    i = pl.program_id(0)
    T = feat_ref.shape[1]

    row_ids = lax.broadcasted_iota(jnp.int32, (T, 1), 0)
    col16 = lax.broadcasted_iota(jnp.int32, (1, PAD), 1)
    colf16 = col16.astype(jnp.float32)
    ones_col = jnp.ones((T, 1), jnp.float32)
    ones_row = jnp.ones((feat_ref.shape[2], 1), jnp.float32).T
    inf = jnp.float32(jnp.inf)

    dlgs = [feat_ref[g] for g in range(GROUP)]
    valids = [row_ids < len_ref[i * GROUP + g] for g in range(GROUP)]

    def seg_mean(g, oh):
        sums = lax.dot_general(oh, dlgs[g], (((0,), (0,)), ((), ())),
                               precision=_HI)
        cnts = lax.dot_general(oh, ones_col, (((0,), (0,)), ((), ())),
                               precision=_HI)
        return sums / jnp.maximum(cnts, 1.0)

    protos = []
    for g in range(GROUP):
        lab = lab_ref[g]
        oh_l = jnp.where((lab == col16) & valids[g], 1.0, 0.0)
        protos.append(seg_mean(g, oh_l))

    def km_iter(_, centers_t):
        new_centers = []
        for g in range(GROUP):
            centers = centers_t[g]
            xc = lax.dot_general(dlgs[g], centers, (((1,), (1,)), ((), ())),
                                 precision=_HI)
            csq = centers * centers
            cc = lax.dot_general(ones_row, csq, (((1,), (1,)), ((), ())),
                                 precision=_HI)
            d2 = jnp.where(col16 < STATE, cc - 2.0 * xc, inf)
            dmin = jnp.min(d2, axis=1, keepdims=True)
            idxmin = jnp.min(jnp.where(d2 <= dmin, colf16, 16.0), axis=1,
                             keepdims=True)
            oh = jnp.where((colf16 == idxmin) & valids[g], 1.0, 0.0)
            new_centers.append(seg_mean(g, oh))
        return tuple(new_centers)

    centers_t = lax.fori_loop(
        0, KM_ITERS, km_iter,
        tuple(dlgs[g][0:PAD, :] for g in range(GROUP)))

    for g in range(GROUP):
        cols = []
        for j in range(STATE):
            diff = protos[g] - centers_t[g][j:j + 1, :]
            cols.append(jnp.sum(diff * diff, axis=1, keepdims=True))
        cols.append(jnp.zeros((PAD, PAD - STATE), jnp.float32))
        d2m = jnp.concatenate(cols, axis=1)
        out_ref[g] = jnp.sqrt(d2m + EPS)


def _tc_distances(features, lengths, labels3):
    B, T, D = features.shape
    return pl.pallas_call(
        _tc_body,
        grid_spec=pltpu.PrefetchScalarGridSpec(
            num_scalar_prefetch=1,
            grid=(B // GROUP,),
            in_specs=[
                pl.BlockSpec((GROUP, T, D), lambda i, s: (i, 0, 0)),
                pl.BlockSpec((GROUP, T, 1), lambda i, s: (i, 0, 0)),
            ],
            out_specs=pl.BlockSpec((GROUP, PAD, PAD), lambda i, s: (i, 0, 0)),
        ),
        out_shape=jax.ShapeDtypeStruct((B, PAD, PAD), jnp.float32),
    )(lengths, features, labels3)


def _make_sc_match(B):
    mesh = plsc.VectorSubcoreMesh(core_axis_name="c", subcore_axis_name="s")

    @functools.partial(
        pl.kernel,
        out_type=jax.ShapeDtypeStruct((B, PAD), jnp.float32),
        mesh=mesh,
        scratch_types=[
            pltpu.VMEM((PAD, PAD), jnp.float32),
            pltpu.VMEM((PAD,), jnp.float32),
            pltpu.SMEM((PAD * PAD,), jnp.float32),
            pltpu.SMEM((PAD * PAD,), jnp.int32),
            pltpu.SMEM((PAD,), jnp.int32),
            pltpu.SMEM((PAD,), jnp.int32),
            pltpu.SMEM((PAD,), jnp.int32),
            pltpu.SMEM((PAD,), jnp.int32),
            pltpu.SMEM((PAD,), jnp.int32),
        ],
    )
    def sc_match(dist_hbm, out_hbm, dist_v, row_v, dist_s, adj_s, p_s, vis_s,
                 node_s, jptr_s, choice_s):
        cid = lax.axis_index("c")
        sid = lax.axis_index("s")

        @pl.when((cid == 0) & (sid < B))
        def _():
            i = sid
            pltpu.sync_copy(dist_hbm.at[i], dist_v)
            lanev = lax.iota(jnp.int32, PAD)
            ninf = jnp.float32(-jnp.inf)

            for r in range(STATE):
                rowr = dist_v[r]
                for c in range(STATE):
                    dist_s[r * PAD + c] = rowr[c]

            def initc(c, _):
                p_s[c] = 0
                choice_s[c] = 0
                return 0

            lax.fori_loop(0, PAD, initc, 0)

            def rowfn(r, _):
                def clearc(c, _):
                    adj_s[r * PAD + c] = 0
                    return 0

                lax.fori_loop(0, PAD, clearc, 0)

                def kpass(k, _):
                    def cscan(c, bst):
                        bv, bi = bst
                        v = dist_s[r * PAD + c]
                        better = (adj_s[r * PAD + c] == 0) & (v > bv)
                        return (jnp.where(better, v, bv),
                                jnp.where(better, c, bi))

                    _, bi = lax.fori_loop(
                        0, STATE, cscan, (ninf, jnp.int32(0)))
                    adj_s[r * PAD + bi] = 1
                    return 0

                lax.fori_loop(0, TOPK, kpass, 0)
                return 0

            lax.fori_loop(0, STATE, rowfn, 0)

            def outer(i1, cnt):
                def clearv(c, _):
                    vis_s[c] = 0
                    return 0

                lax.fori_loop(0, PAD, clearv, 0)
                node_s[0] = i1
                jptr_s[0] = 0

                def step(t, st):
                    def dead():
                        return st

                    def live():
                        return _dfs_step(st)

                    return lax.cond(st[1], dead, live)

                def _dfs_step(st):
                    depth, done, succ = st
                    ii = node_s[depth]
                    j = jptr_s[depth]
                    exhausted = j >= STATE
                    jc = jnp.minimum(j, STATE - 1)
                    adjv = adj_s[(ii - 1) * PAD + jc] != 0
                    can = adjv & (vis_s[jc] == 0) & jnp.logical_not(exhausted)
                    pjc = p_s[jc]
                    free = can & (pjc == 0)
                    descend = can & (pjc != 0)
                    skip = jnp.logical_not(can) & jnp.logical_not(exhausted)
                    vis_s[jc] = jnp.where(can, 1, vis_s[jc])
                    choice_s[depth] = jnp.where(can, jc, choice_s[depth])

                    def unwind(d, _):
                        pred = free & (d <= depth)
                        idx = choice_s[d]
                        p_s[idx] = jnp.where(pred, node_s[d], p_s[idx])
                        return 0

                    lax.fori_loop(0, STATE, unwind, 0)
                    jptr_s[depth] = jnp.where(skip | descend, jc + 1, j)
                    dp1 = depth + 1
                    node_s[dp1] = jnp.where(descend, pjc, node_s[dp1])
                    jptr_s[dp1] = jnp.where(descend, 0, jptr_s[dp1])
                    depth2 = (depth + jnp.where(descend, 1, 0)
                              - jnp.where(exhausted, 1, 0))
                    done2 = done | free | (exhausted & (depth == 0))
                    return (depth2, done2, succ | free)

                st = lax.fori_loop(
                    0, 128, step,
                    (jnp.int32(0), jnp.bool_(False), jnp.bool_(False)))
                return cnt + st[2].astype(jnp.int32)

            cnt = lax.fori_loop(1, STATE + 1, outer, jnp.int32(0))

            def jloop(j, acc):
                pj = p_s[j]
                idx = jnp.maximum(pj - 1, 0)
                v = dist_s[idx * PAD + j]
                return acc + jnp.where(pj > 0, v, jnp.float32(0.0))

            loss = lax.fori_loop(0, STATE, jloop, jnp.float32(0.0))
            cntf = jnp.maximum(cnt, 1).astype(jnp.float32)
            row_v[...] = jnp.where(lanev == 0, loss,
                                   jnp.where(lanev == 1, cntf,
                                             jnp.float32(0.0)))
            pltpu.sync_copy(row_v, out_hbm.at[i])

    return sc_match


def kernel(features, dialogue_lengths, labels):
    B, T, D = features.shape
    lengths = dialogue_lengths.astype(jnp.int32)
    labels3 = labels.astype(jnp.int32).reshape(B, T, 1)
    dist = _tc_distances(features, lengths, labels3)
    out = _make_sc_match(B)(dist)
    return jnp.mean(out[:, 0] / out[:, 1])


# speedup vs baseline: 1.4058x; 1.4058x over previous
import functools

import jax
import jax.numpy as jnp
from jax import lax
from jax.experimental import pallas as pl
from jax.experimental.pallas import tpu as pltpu
from jax.experimental.pallas import tpu_sc as plsc

STATE = 10
KM_ITERS = 10
TOPK = 3
PAD = 16
EPS = 1e-12
_HI = jax.lax.Precision.HIGHEST


def _split3(x):
    hi = x.astype(jnp.bfloat16)
    r1 = x - hi.astype(jnp.float32)
    mid = r1.astype(jnp.bfloat16)
    lo = (r1 - mid.astype(jnp.float32)).astype(jnp.bfloat16)
    return hi, mid, lo


def _tc_body(len_ref, feat_ref, lab_ref, out_ref):
    i = pl.program_id(0)
    L = len_ref[i]
    dlg = feat_ref[0]
    lab = lab_ref[0]
    T = dlg.shape[0]

    row_ids = lax.broadcasted_iota(jnp.int32, (T, 1), 0)
    valid = row_ids < L
    col16 = lax.broadcasted_iota(jnp.int32, (1, PAD), 1)
    colf16 = col16.astype(jnp.float32)
    ones_col = jnp.ones((T, 1), jnp.bfloat16)
    ones_row = jnp.ones((1, dlg.shape[1]), jnp.float32)
    inf = jnp.float32(jnp.inf)

    dhi, dmid, dlo = _split3(dlg)

    def mmT(a, b):
        return lax.dot_general(a, b, (((0,), (0,)), ((), ())),
                               preferred_element_type=jnp.float32)

    def mm(a, b):
        return lax.dot_general(a, b, (((1,), (1,)), ((), ())),
                               preferred_element_type=jnp.float32)

    def seg_mean(oh):
        ohb = oh.astype(jnp.bfloat16)
        sums = mmT(ohb, dhi) + mmT(ohb, dmid) + mmT(ohb, dlo)
        cnts = mmT(ohb, ones_col)
        return sums / jnp.maximum(cnts, 1.0)

    oh_l = jnp.where((lab == col16) & valid, 1.0, 0.0)
    protos = seg_mean(oh_l)

    def km_iter(_, centers):
        chi, cmid, clo = _split3(centers)
        xc = (mm(dhi, chi) + mm(dhi, cmid) + mm(dmid, chi)
              + mm(dhi, clo) + mm(dlo, chi) + mm(dmid, cmid))
        csq = centers * centers
        cc = lax.dot_general(ones_row, csq, (((1,), (1,)), ((), ())),
                             precision=_HI)
        d2 = jnp.where(col16 < STATE, cc - 2.0 * xc, inf)
        dmin = jnp.min(d2, axis=1, keepdims=True)
        idxmin = jnp.min(jnp.where(d2 <= dmin, colf16, 16.0), axis=1,
                         keepdims=True)
        oh = jnp.where((colf16 == idxmin) & valid, 1.0, 0.0)
        return seg_mean(oh)

    centers = lax.fori_loop(0, KM_ITERS, km_iter, dlg[0:PAD, :])

    cols = []
    for j in range(STATE):
        diff = protos - centers[j:j + 1, :]
        cols.append(jnp.sum(diff * diff, axis=1, keepdims=True))
    cols.append(jnp.zeros((PAD, PAD - STATE), jnp.float32))
    d2m = jnp.concatenate(cols, axis=1)
    out_ref[0] = jnp.sqrt(d2m + EPS)


def _tc_distances(features, lengths, labels3):
    B, T, D = features.shape
    return pl.pallas_call(
        _tc_body,
        grid_spec=pltpu.PrefetchScalarGridSpec(
            num_scalar_prefetch=1,
            grid=(B,),
            in_specs=[
                pl.BlockSpec((1, T, D), lambda i, s: (i, 0, 0)),
                pl.BlockSpec((1, T, 1), lambda i, s: (i, 0, 0)),
            ],
            out_specs=pl.BlockSpec((1, PAD, PAD), lambda i, s: (i, 0, 0)),
        ),
        out_shape=jax.ShapeDtypeStruct((B, PAD, PAD), jnp.float32),
    )(lengths, features, labels3)


def _make_sc_match(B):
    mesh = plsc.VectorSubcoreMesh(core_axis_name="c", subcore_axis_name="s")

    @functools.partial(
        pl.kernel,
        out_type=jax.ShapeDtypeStruct((B, PAD), jnp.float32),
        mesh=mesh,
        scratch_types=[
            pltpu.VMEM((PAD, PAD), jnp.float32),
            pltpu.VMEM((PAD,), jnp.float32),
            pltpu.SMEM((PAD * PAD,), jnp.float32),
            pltpu.SMEM((PAD * PAD,), jnp.int32),
            pltpu.SMEM((PAD,), jnp.int32),
            pltpu.SMEM((PAD,), jnp.int32),
            pltpu.SMEM((PAD,), jnp.int32),
            pltpu.SMEM((PAD,), jnp.int32),
            pltpu.SMEM((PAD,), jnp.int32),
        ],
    )
    def sc_match(dist_hbm, out_hbm, dist_v, row_v, dist_s, adj_s, p_s, vis_s,
                 node_s, jptr_s, choice_s):
        cid = lax.axis_index("c")
        sid = lax.axis_index("s")

        @pl.when((cid == 0) & (sid < B))
        def _():
            i = sid
            pltpu.sync_copy(dist_hbm.at[i], dist_v)
            lanev = lax.iota(jnp.int32, PAD)
            ninf = jnp.float32(-jnp.inf)

            for r in range(STATE):
                rowr = dist_v[r]
                for c in range(STATE):
                    dist_s[r * PAD + c] = rowr[c]

            def initc(c, _):
                p_s[c] = 0
                choice_s[c] = 0
                return 0

            lax.fori_loop(0, PAD, initc, 0)

            def rowfn(r, _):
                def clearc(c, _):
                    adj_s[r * PAD + c] = 0
                    return 0

                lax.fori_loop(0, PAD, clearc, 0)

                def kpass(k, _):
                    def cscan(c, bst):
                        bv, bi = bst
                        v = dist_s[r * PAD + c]
                        better = (adj_s[r * PAD + c] == 0) & (v > bv)
                        return (jnp.where(better, v, bv),
                                jnp.where(better, c, bi))

                    _, bi = lax.fori_loop(
                        0, STATE, cscan, (ninf, jnp.int32(0)))
                    adj_s[r * PAD + bi] = 1
                    return 0

                lax.fori_loop(0, TOPK, kpass, 0)
                return 0

            lax.fori_loop(0, STATE, rowfn, 0)

            def outer(i1, cnt):
                def clearv(c, _):
                    vis_s[c] = 0
                    return 0

                lax.fori_loop(0, PAD, clearv, 0)
                node_s[0] = i1
                jptr_s[0] = 0

                def step(t, st):
                    def dead():
                        return st

                    def live():
                        return _dfs_step(st)

                    return lax.cond(st[1], dead, live)

                def _dfs_step(st):
                    depth, done, succ = st
                    ii = node_s[depth]
                    j = jptr_s[depth]
                    exhausted = j >= STATE
                    jc = jnp.minimum(j, STATE - 1)
                    adjv = adj_s[(ii - 1) * PAD + jc] != 0
                    can = adjv & (vis_s[jc] == 0) & jnp.logical_not(exhausted)
                    pjc = p_s[jc]
                    free = can & (pjc == 0)
                    descend = can & (pjc != 0)
                    skip = jnp.logical_not(can) & jnp.logical_not(exhausted)
                    vis_s[jc] = jnp.where(can, 1, vis_s[jc])
                    choice_s[depth] = jnp.where(can, jc, choice_s[depth])

                    def unwind(d, _):
                        pred = free & (d <= depth)
                        idx = choice_s[d]
                        p_s[idx] = jnp.where(pred, node_s[d], p_s[idx])
                        return 0

                    lax.fori_loop(0, STATE, unwind, 0)
                    jptr_s[depth] = jnp.where(skip | descend, jc + 1, j)
                    dp1 = depth + 1
                    node_s[dp1] = jnp.where(descend, pjc, node_s[dp1])
                    jptr_s[dp1] = jnp.where(descend, 0, jptr_s[dp1])
                    depth2 = (depth + jnp.where(descend, 1, 0)
                              - jnp.where(exhausted, 1, 0))
                    done2 = done | free | (exhausted & (depth == 0))
                    return (depth2, done2, succ | free)

                st = lax.fori_loop(
                    0, 128, step,
                    (jnp.int32(0), jnp.bool_(False), jnp.bool_(False)))
                return cnt + st[2].astype(jnp.int32)

            cnt = lax.fori_loop(1, STATE + 1, outer, jnp.int32(0))

            def jloop(j, acc):
                pj = p_s[j]
                idx = jnp.maximum(pj - 1, 0)
                v = dist_s[idx * PAD + j]
                return acc + jnp.where(pj > 0, v, jnp.float32(0.0))

            loss = lax.fori_loop(0, STATE, jloop, jnp.float32(0.0))
            cntf = jnp.maximum(cnt, 1).astype(jnp.float32)
            row_v[...] = jnp.where(lanev == 0, loss,
                                   jnp.where(lanev == 1, cntf,
                                             jnp.float32(0.0)))
            pltpu.sync_copy(row_v, out_hbm.at[i])

    return sc_match


def kernel(features, dialogue_lengths, labels):
    B, T, D = features.shape
    lengths = dialogue_lengths.astype(jnp.int32)
    labels3 = labels.astype(jnp.int32).reshape(B, T, 1)
    dist = _tc_distances(features, lengths, labels3)
    out = _make_sc_match(B)(dist)
    return jnp.mean(out[:, 0] / out[:, 1])
